# baseline (speedup 1.0000x reference)
import jax
import jax.numpy as jnp
from jax import lax
from jax.experimental import pallas as pl
from jax.experimental.pallas import tpu as pltpu
from jax.experimental.pallas import tpu_sc as plsc

DIM = 32
BATCH = 16384
SB = 2
NBUF = 4


def _sc_body(u_idx_hbm, i_idx_hbm, j_idx_hbm, ut, it,
             pos_out, neg_out,
             idx_u, idx_i, idx_j, gran_a, gran_b, gran_c, gran_d,
             pos_v, neg_v, sem):
    info = plsc.get_sparse_core_info()
    nc = info.num_cores
    nw = nc * info.num_subcores
    b_per_w = BATCH // nw

    wid = lax.axis_index("s") * nc + lax.axis_index("c")
    base = wid * b_per_w

    pltpu.sync_copy(u_idx_hbm.at[pl.ds(wid * 4, 4)], idx_u)
    pltpu.sync_copy(i_idx_hbm.at[pl.ds(wid * 4, 4)], idx_i)
    pltpu.sync_copy(j_idx_hbm.at[pl.ds(wid * 4, 4)], idx_j)

    lanes16 = lax.iota(jnp.int32, 16)
    c0 = lanes16
    c1 = lanes16 + 16
    bufs = (gran_a, gran_b, gran_c, gran_d)
    n_sb = 16 // SB

    def fire(ivs, sb, buf):
        iu, ii, ij = ivs
        copies = []
        for k in range(SB):
            l = sb * SB + k
            for t, (iv, src) in enumerate(((iu, ut), (ii, it), (ij, it))):
                r = iv[l]
                rblk = pl.multiple_of((r >> 7) << 7, 128)
                for b in range(4):
                    copies.append(pltpu.async_copy(
                        src.at[pl.ds(b * 8, 8), pl.ds(rblk, 128)],
                        buf.at[k * 3 + t, pl.ds(b * 8, 8), :], sem))
        return copies

    def extract(ivs, sb, buf, acc_p, acc_n):
        iu, ii, ij = ivs
        for k in range(SB):
            l = sb * SB + k
            qu = jnp.full((16,), k * 3 + 0, jnp.int32)
            qi = jnp.full((16,), k * 3 + 1, jnp.int32)
            qj = jnp.full((16,), k * 3 + 2, jnp.int32)
            lu = jnp.full((16,), iu[l] & 127, jnp.int32)
            li = jnp.full((16,), ii[l] & 127, jnp.int32)
            lj = jnp.full((16,), ij[l] & 127, jnp.int32)
            u0 = plsc.load_gather(buf, [qu, c0, lu])
            u1 = plsc.load_gather(buf, [qu, c1, lu])
            i0 = plsc.load_gather(buf, [qi, c0, li])
            i1 = plsc.load_gather(buf, [qi, c1, li])
            j0 = plsc.load_gather(buf, [qj, c0, lj])
            j1 = plsc.load_gather(buf, [qj, c1, lj])
            ps = jnp.sum(u0 * i0 + u1 * i1)
            ns = jnp.sum(u0 * j0 + u1 * j1)
            sel = lanes16 == l
            acc_p = jnp.where(sel, ps, acc_p)
            acc_n = jnp.where(sel, ns, acc_n)
        return acc_p, acc_n

    def body16(g, _):
        d0 = g // 8
        m = (g % 8) * 16
        ivs = (idx_u[d0, pl.ds(m, 16)],
               idx_i[d0, pl.ds(m, 16)],
               idx_j[d0, pl.ds(m, 16)])

        acc_p = jnp.zeros((16,), jnp.float32)
        acc_n = jnp.zeros((16,), jnp.float32)
        pending = [fire(ivs, sb, bufs[sb]) for sb in range(NBUF - 1)]
        for sb in range(n_sb):
            if sb + NBUF - 1 < n_sb:
                pending.append(
                    fire(ivs, sb + NBUF - 1, bufs[(sb + NBUF - 1) % NBUF]))
            for cp in pending.pop(0):
                cp.wait()
            acc_p, acc_n = extract(ivs, sb, bufs[sb % NBUF], acc_p, acc_n)
        pos_v[pl.ds(g * 16, 16)] = acc_p
        neg_v[pl.ds(g * 16, 16)] = acc_n
        return 0

    lax.fori_loop(0, b_per_w // 16, body16, 0)

    pltpu.sync_copy(pos_v, pos_out.at[pl.ds(base, b_per_w)])
    pltpu.sync_copy(neg_v, neg_out.at[pl.ds(base, b_per_w)])


def kernel(batch_user, batch_pos_item, batch_neg_item, user_emb, item_emb):
    info = plsc.get_sparse_core_info()
    nw = info.num_cores * info.num_subcores
    b_per_w = BATCH // nw

    ut = user_emb.T
    it = item_emb.T

    u_idx = batch_user.reshape(BATCH // 128, 128)
    i_idx = batch_pos_item.reshape(BATCH // 128, 128)
    j_idx = batch_neg_item.reshape(BATCH // 128, 128)

    mesh = plsc.VectorSubcoreMesh(core_axis_name="c", subcore_axis_name="s")
    run = pl.kernel(
        _sc_body,
        mesh=mesh,
        compiler_params=pltpu.CompilerParams(
            needs_layout_passes=False, use_tc_tiling_on_sc=True),
        out_type=(
            jax.ShapeDtypeStruct((BATCH,), jnp.float32),
            jax.ShapeDtypeStruct((BATCH,), jnp.float32),
        ),
        scratch_types=[
            pltpu.VMEM((4, 128), jnp.int32),
            pltpu.VMEM((4, 128), jnp.int32),
            pltpu.VMEM((4, 128), jnp.int32),
            pltpu.VMEM((3 * SB, DIM, 128), jnp.float32),
            pltpu.VMEM((3 * SB, DIM, 128), jnp.float32),
            pltpu.VMEM((3 * SB, DIM, 128), jnp.float32),
            pltpu.VMEM((3 * SB, DIM, 128), jnp.float32),
            pltpu.VMEM((b_per_w,), jnp.float32),
            pltpu.VMEM((b_per_w,), jnp.float32),
            pltpu.SemaphoreType.DMA,
        ],
    )
    pos, neg = run(u_idx, i_idx, j_idx, ut, it)
    return (pos.reshape(BATCH, 1), neg.reshape(BATCH, 1))


# speedup vs baseline: 1.0083x; 1.0083x over previous
import jax
import jax.numpy as jnp
from jax import lax
from jax.experimental import pallas as pl
from jax.experimental.pallas import tpu as pltpu
from jax.experimental.pallas import tpu_sc as plsc

DIM = 32
BATCH = 16384
SB = 2
NBUF = 4


def _sc_body(u_idx_hbm, i_idx_hbm, j_idx_hbm, ut, it,
             pos_out, neg_out,
             idx_u, idx_i, idx_j, gran_a, gran_b, gran_c, gran_d,
             pos_v, neg_v, sem):
    info = plsc.get_sparse_core_info()
    nc = info.num_cores
    nw = nc * info.num_subcores
    b_per_w = BATCH // nw

    wid = lax.axis_index("s") * nc + lax.axis_index("c")
    base = wid * b_per_w

    pltpu.sync_copy(u_idx_hbm.at[pl.ds(wid * 4, 4)], idx_u)
    pltpu.sync_copy(i_idx_hbm.at[pl.ds(wid * 4, 4)], idx_i)
    pltpu.sync_copy(j_idx_hbm.at[pl.ds(wid * 4, 4)], idx_j)

    lanes16 = lax.iota(jnp.int32, 16)
    c0 = lanes16
    c1 = lanes16 + 16
    bufs = (gran_a, gran_b, gran_c, gran_d)
    n_sb = 16 // SB

    def fire(ivs, sb, buf):
        iu, ii, ij = ivs
        copies = []
        for k in range(SB):
            l = sb * SB + k
            for t, (iv, src) in enumerate(((iu, ut), (ii, it), (ij, it))):
                r = iv[l]
                rblk = pl.multiple_of((r >> 7) << 7, 128)
                copies.append(pltpu.async_copy(
                    src.at[:, pl.ds(rblk, 128)], buf.at[k * 3 + t], sem))
        return copies

    def extract(ivs, sb, buf, acc_p, acc_n):
        iu, ii, ij = ivs
        for k in range(SB):
            l = sb * SB + k
            qu = jnp.full((16,), k * 3 + 0, jnp.int32)
            qi = jnp.full((16,), k * 3 + 1, jnp.int32)
            qj = jnp.full((16,), k * 3 + 2, jnp.int32)
            lu = jnp.full((16,), iu[l] & 127, jnp.int32)
            li = jnp.full((16,), ii[l] & 127, jnp.int32)
            lj = jnp.full((16,), ij[l] & 127, jnp.int32)
            u0 = plsc.load_gather(buf, [qu, c0, lu])
            u1 = plsc.load_gather(buf, [qu, c1, lu])
            i0 = plsc.load_gather(buf, [qi, c0, li])
            i1 = plsc.load_gather(buf, [qi, c1, li])
            j0 = plsc.load_gather(buf, [qj, c0, lj])
            j1 = plsc.load_gather(buf, [qj, c1, lj])
            ps = jnp.sum(u0 * i0 + u1 * i1)
            ns = jnp.sum(u0 * j0 + u1 * j1)
            sel = lanes16 == l
            acc_p = jnp.where(sel, ps, acc_p)
            acc_n = jnp.where(sel, ns, acc_n)
        return acc_p, acc_n

    def body16(g, _):
        d0 = g // 8
        m = (g % 8) * 16
        ivs = (idx_u[d0, pl.ds(m, 16)],
               idx_i[d0, pl.ds(m, 16)],
               idx_j[d0, pl.ds(m, 16)])

        acc_p = jnp.zeros((16,), jnp.float32)
        acc_n = jnp.zeros((16,), jnp.float32)
        pending = [fire(ivs, sb, bufs[sb]) for sb in range(NBUF - 1)]
        for sb in range(n_sb):
            if sb + NBUF - 1 < n_sb:
                pending.append(
                    fire(ivs, sb + NBUF - 1, bufs[(sb + NBUF - 1) % NBUF]))
            for cp in pending.pop(0):
                cp.wait()
            acc_p, acc_n = extract(ivs, sb, bufs[sb % NBUF], acc_p, acc_n)
        pos_v[pl.ds(g * 16, 16)] = acc_p
        neg_v[pl.ds(g * 16, 16)] = acc_n
        return 0

    lax.fori_loop(0, b_per_w // 16, body16, 0)

    pltpu.sync_copy(pos_v, pos_out.at[pl.ds(base, b_per_w)])
    pltpu.sync_copy(neg_v, neg_out.at[pl.ds(base, b_per_w)])


def kernel(batch_user, batch_pos_item, batch_neg_item, user_emb, item_emb):
    info = plsc.get_sparse_core_info()
    nw = info.num_cores * info.num_subcores
    b_per_w = BATCH // nw

    ut = user_emb.T
    it = item_emb.T

    u_idx = batch_user.reshape(BATCH // 128, 128)
    i_idx = batch_pos_item.reshape(BATCH // 128, 128)
    j_idx = batch_neg_item.reshape(BATCH // 128, 128)

    mesh = plsc.VectorSubcoreMesh(core_axis_name="c", subcore_axis_name="s")
    run = pl.kernel(
        _sc_body,
        mesh=mesh,
        compiler_params=pltpu.CompilerParams(
            needs_layout_passes=False, use_tc_tiling_on_sc=True),
        out_type=(
            jax.ShapeDtypeStruct((BATCH,), jnp.float32),
            jax.ShapeDtypeStruct((BATCH,), jnp.float32),
        ),
        scratch_types=[
            pltpu.VMEM((4, 128), jnp.int32),
            pltpu.VMEM((4, 128), jnp.int32),
            pltpu.VMEM((4, 128), jnp.int32),
            pltpu.VMEM((3 * SB, DIM, 128), jnp.float32),
            pltpu.VMEM((3 * SB, DIM, 128), jnp.float32),
            pltpu.VMEM((3 * SB, DIM, 128), jnp.float32),
            pltpu.VMEM((3 * SB, DIM, 128), jnp.float32),
            pltpu.VMEM((b_per_w,), jnp.float32),
            pltpu.VMEM((b_per_w,), jnp.float32),
            pltpu.SemaphoreType.DMA,
        ],
    )
    pos, neg = run(u_idx, i_idx, j_idx, ut, it)
    return (pos.reshape(BATCH, 1), neg.reshape(BATCH, 1))


# speedup vs baseline: 1.0095x; 1.0012x over previous
import jax
import jax.numpy as jnp
from jax import lax
from jax.experimental import pallas as pl
from jax.experimental.pallas import tpu as pltpu
from jax.experimental.pallas import tpu_sc as plsc

DIM = 32
BATCH = 16384
SB = 1
NBUF = 8


def _sc_body(u_idx_hbm, i_idx_hbm, j_idx_hbm, ut, it,
             pos_out, neg_out,
             idx_u, idx_i, idx_j, gran_a, gran_b, gran_c, gran_d,
             gran_e, gran_f, gran_g, gran_h,
             pos_v, neg_v, sem):
    info = plsc.get_sparse_core_info()
    nc = info.num_cores
    nw = nc * info.num_subcores
    b_per_w = BATCH // nw

    wid = lax.axis_index("s") * nc + lax.axis_index("c")
    base = wid * b_per_w

    pltpu.sync_copy(u_idx_hbm.at[pl.ds(wid * 4, 4)], idx_u)
    pltpu.sync_copy(i_idx_hbm.at[pl.ds(wid * 4, 4)], idx_i)
    pltpu.sync_copy(j_idx_hbm.at[pl.ds(wid * 4, 4)], idx_j)

    lanes16 = lax.iota(jnp.int32, 16)
    c0 = lanes16
    c1 = lanes16 + 16
    bufs = (gran_a, gran_b, gran_c, gran_d, gran_e, gran_f, gran_g, gran_h)
    n_sb = 16 // SB

    def fire(ivs, sb, buf):
        iu, ii, ij = ivs
        copies = []
        for k in range(SB):
            l = sb * SB + k
            for t, (iv, src) in enumerate(((iu, ut), (ii, it), (ij, it))):
                r = iv[l]
                rblk = pl.multiple_of((r >> 7) << 7, 128)
                copies.append(pltpu.async_copy(
                    src.at[:, pl.ds(rblk, 128)], buf.at[k * 3 + t], sem))
        return copies

    def extract(ivs, sb, buf, acc_p, acc_n):
        iu, ii, ij = ivs
        for k in range(SB):
            l = sb * SB + k
            qu = jnp.full((16,), k * 3 + 0, jnp.int32)
            qi = jnp.full((16,), k * 3 + 1, jnp.int32)
            qj = jnp.full((16,), k * 3 + 2, jnp.int32)
            lu = jnp.full((16,), iu[l] & 127, jnp.int32)
            li = jnp.full((16,), ii[l] & 127, jnp.int32)
            lj = jnp.full((16,), ij[l] & 127, jnp.int32)
            u0 = plsc.load_gather(buf, [qu, c0, lu])
            u1 = plsc.load_gather(buf, [qu, c1, lu])
            i0 = plsc.load_gather(buf, [qi, c0, li])
            i1 = plsc.load_gather(buf, [qi, c1, li])
            j0 = plsc.load_gather(buf, [qj, c0, lj])
            j1 = plsc.load_gather(buf, [qj, c1, lj])
            ps = jnp.sum(u0 * i0 + u1 * i1)
            ns = jnp.sum(u0 * j0 + u1 * j1)
            sel = lanes16 == l
            acc_p = jnp.where(sel, ps, acc_p)
            acc_n = jnp.where(sel, ns, acc_n)
        return acc_p, acc_n

    def body16(g, _):
        d0 = g // 8
        m = (g % 8) * 16
        ivs = (idx_u[d0, pl.ds(m, 16)],
               idx_i[d0, pl.ds(m, 16)],
               idx_j[d0, pl.ds(m, 16)])

        acc_p = jnp.zeros((16,), jnp.float32)
        acc_n = jnp.zeros((16,), jnp.float32)
        pending = [fire(ivs, sb, bufs[sb]) for sb in range(NBUF - 1)]
        for sb in range(n_sb):
            if sb + NBUF - 1 < n_sb:
                pending.append(
                    fire(ivs, sb + NBUF - 1, bufs[(sb + NBUF - 1) % NBUF]))
            for cp in pending.pop(0):
                cp.wait()
            acc_p, acc_n = extract(ivs, sb, bufs[sb % NBUF], acc_p, acc_n)
        pos_v[pl.ds(g * 16, 16)] = acc_p
        neg_v[pl.ds(g * 16, 16)] = acc_n
        return 0

    lax.fori_loop(0, b_per_w // 16, body16, 0)

    pltpu.sync_copy(pos_v, pos_out.at[pl.ds(base, b_per_w)])
    pltpu.sync_copy(neg_v, neg_out.at[pl.ds(base, b_per_w)])


def kernel(batch_user, batch_pos_item, batch_neg_item, user_emb, item_emb):
    info = plsc.get_sparse_core_info()
    nw = info.num_cores * info.num_subcores
    b_per_w = BATCH // nw

    ut = user_emb.T
    it = item_emb.T

    u_idx = batch_user.reshape(BATCH // 128, 128)
    i_idx = batch_pos_item.reshape(BATCH // 128, 128)
    j_idx = batch_neg_item.reshape(BATCH // 128, 128)

    mesh = plsc.VectorSubcoreMesh(core_axis_name="c", subcore_axis_name="s")
    run = pl.kernel(
        _sc_body,
        mesh=mesh,
        compiler_params=pltpu.CompilerParams(
            needs_layout_passes=False, use_tc_tiling_on_sc=True),
        out_type=(
            jax.ShapeDtypeStruct((BATCH,), jnp.float32),
            jax.ShapeDtypeStruct((BATCH,), jnp.float32),
        ),
        scratch_types=[
            pltpu.VMEM((4, 128), jnp.int32),
            pltpu.VMEM((4, 128), jnp.int32),
            pltpu.VMEM((4, 128), jnp.int32),
            pltpu.VMEM((3 * SB, DIM, 128), jnp.float32),
            pltpu.VMEM((3 * SB, DIM, 128), jnp.float32),
            pltpu.VMEM((3 * SB, DIM, 128), jnp.float32),
            pltpu.VMEM((3 * SB, DIM, 128), jnp.float32),
            pltpu.VMEM((3 * SB, DIM, 128), jnp.float32),
            pltpu.VMEM((3 * SB, DIM, 128), jnp.float32),
            pltpu.VMEM((3 * SB, DIM, 128), jnp.float32),
            pltpu.VMEM((3 * SB, DIM, 128), jnp.float32),
            pltpu.VMEM((b_per_w,), jnp.float32),
            pltpu.VMEM((b_per_w,), jnp.float32),
            pltpu.SemaphoreType.DMA,
        ],
    )
    pos, neg = run(u_idx, i_idx, j_idx, ut, it)
    return (pos.reshape(BATCH, 1), neg.reshape(BATCH, 1))
